# Initial kernel scaffold; baseline (speedup 1.0000x reference)
import jax
import jax.numpy as jnp
from jax.experimental import pallas as pl


def kernel(feat, edge_index, edge_weight, W_neigh, W_dst, W_self, W_edge, b_edge, W_prj_src, b_prj_src, W_prj_dst, b_prj_dst, W_prj_edge, b_prj_edge, W_att, b_att, prelu_alpha, out_bias):
    raise NotImplementedError("write your pallas kernel here")


# speedup vs baseline: 1.0545x; 1.0545x over previous
import jax
import jax.numpy as jnp
from jax.experimental import pallas as pl
from jax.experimental.pallas import tpu as pltpu


def _mm_bias_body(x_ref, w_ref, b_ref, o_ref):
    o_ref[...] = jnp.dot(x_ref[...], w_ref[...],
                         preferred_element_type=jnp.float32) + b_ref[...]


def _mm(x, w, b, block_rows):
    M, K = x.shape
    _, Nc = w.shape
    return pl.pallas_call(
        _mm_bias_body,
        grid=(M // block_rows,),
        in_specs=[pl.BlockSpec((block_rows, K), lambda i: (i, 0)),
                  pl.BlockSpec((K, Nc), lambda i: (0, 0)),
                  pl.BlockSpec((1, Nc), lambda i: (0, 0))],
        out_specs=pl.BlockSpec((block_rows, Nc), lambda i: (i, 0)),
        out_shape=jax.ShapeDtypeStruct((M, Nc), jnp.float32),
    )(x, w, b)


def _final_body(s_ref, hd_ref, acc_ref, w_ref, o_ref):
    prod = hd_ref[...] * acc_ref[...]
    o_ref[...] = s_ref[...] + jnp.dot(prod, w_ref[...],
                                      preferred_element_type=jnp.float32)


def _final(self_out, h_dst, acc, w_neigh_t, block_rows):
    M, D = self_out.shape
    return pl.pallas_call(
        _final_body,
        grid=(M // block_rows,),
        in_specs=[pl.BlockSpec((block_rows, D), lambda i: (i, 0)),
                  pl.BlockSpec((block_rows, D), lambda i: (i, 0)),
                  pl.BlockSpec((block_rows, D), lambda i: (i, 0)),
                  pl.BlockSpec((D, D), lambda i: (0, 0))],
        out_specs=pl.BlockSpec((block_rows, D), lambda i: (i, 0)),
        out_shape=jax.ShapeDtypeStruct((M, D), jnp.float32),
    )(self_out, h_dst, acc, w_neigh_t)


def kernel(feat, edge_index, edge_weight, W_neigh, W_dst, W_self, W_edge, b_edge,
           W_prj_src, b_prj_src, W_prj_dst, b_prj_dst, W_prj_edge, b_prj_edge,
           W_att, b_att, prelu_alpha, out_bias):
    src = edge_index[0]
    dst = edge_index[1]
    n = feat.shape[0]
    d = feat.shape[1]

    Wn = jnp.concatenate([W_prj_src.T, W_prj_dst.T, W_dst.T, W_self.T], axis=1)
    bn = jnp.concatenate([b_prj_src, b_prj_dst,
                          jnp.zeros_like(b_prj_src), out_bias])[None, :]
    X = _mm(feat, Wn, bn, 2000)
    hw_src = X[:, 0 * d:1 * d]
    hw_dst = X[:, 1 * d:2 * d]
    h_dst = X[:, 2 * d:3 * d]
    self_out = X[:, 3 * d:4 * d]

    We = jnp.concatenate([W_prj_edge.T, W_edge.T], axis=1)
    be = jnp.concatenate([b_prj_edge, b_edge])[None, :]
    EWE = _mm(edge_weight, We, be, 2000)
    ew = EWE[:, :d]
    e = EWE[:, d:]

    w = hw_src[src] + hw_dst[dst] + ew
    w = jnp.where(w >= 0, w, prelu_alpha * w)
    w = (w @ W_att.T + b_att)[:, 0]
    m = jax.ops.segment_max(w, dst, num_segments=n)
    m = jnp.where(jnp.isfinite(m), m, 0.0)
    ex = jnp.exp(w - m[dst])
    ssum = jax.ops.segment_sum(ex, dst, num_segments=n)
    a = ex / ssum[dst]
    l = a[:, None] * e * feat[src]
    acc = jax.ops.segment_sum(l, dst, num_segments=n)

    return _final(self_out, h_dst, acc, W_neigh.T, 2000)
